# baseline (speedup 1.0000x reference)
import functools

import jax
import jax.numpy as jnp
from jax.experimental import pallas as pl
from jax.experimental.pallas import tpu as pltpu

_B = 32
_P = 4096
_K = 512
_D = 64


def _shift_down(z):
    c = z.shape[1]
    return jnp.concatenate([jnp.zeros((1, c), z.dtype), z[:-1, :]], axis=0)


def _shift_up(z):
    c = z.shape[1]
    return jnp.concatenate([z[1:, :], jnp.zeros((1, c), z.dtype)], axis=0)


def _vqvae_body(x_ref, w1b_ref, b1r_ref, w2b_ref, b2r_ref, w3b_ref, b3r_ref,
                w4t_ref, b4r_ref, wpt_ref, bpr_ref,
                et2_ref, e2r_ref, e_ref, wd1b_ref, bd1r_ref,
                wt1b_ref, bt1r_ref, wt2b_ref, bt2r_ref,
                y_ref, idx_ref, lp_ref):
    f32 = jnp.float32
    bf16 = jnp.bfloat16
    dot = functools.partial(jnp.dot, preferred_element_type=f32)

    z1 = jnp.maximum(dot(x_ref[0], w1b_ref[...]) + b1r_ref[...], 0.0).astype(bf16)

    a = z1[:, :_D]
    b = z1[:, _D:]
    z_cat = jnp.concatenate([_shift_down(b), a, b, _shift_up(a)], axis=1)
    z2 = jnp.maximum(dot(z_cat, w2b_ref[...]) + b2r_ref[...], 0.0).astype(bf16)

    z_cat = jnp.concatenate([_shift_down(z2), z2, _shift_up(z2)], axis=1)
    z3 = jnp.maximum(dot(z_cat, w3b_ref[...]) + b3r_ref[...], 0.0).astype(bf16)

    z4 = (dot(z3, w4t_ref[...]) + b4r_ref[...]).astype(bf16)
    flat = dot(z4, wpt_ref[...]) + bpr_ref[...]

    flat2 = jnp.sum(flat * flat, axis=1, keepdims=True)
    dist = (flat2 + e2r_ref[...]) - dot(flat, et2_ref[...])
    mval = jnp.min(dist, axis=1, keepdims=True)
    colid = jax.lax.broadcasted_iota(jnp.int32, (_P, _K), 1)
    idx = jnp.min(jnp.where(dist == mval, colid, _K), axis=1, keepdims=True)
    onehot = (colid == idx).astype(bf16)
    q = dot(onehot, e_ref[...])
    idx_ref[0] = idx

    diff = q - flat
    lp_ref[0] = jnp.sum(diff * diff, axis=0, keepdims=True)

    qb = q.astype(bf16)
    q_cat = jnp.concatenate([_shift_down(qb), qb, _shift_up(qb)], axis=1)
    h = jnp.maximum(dot(q_cat, wd1b_ref[...]) + bd1r_ref[...], 0.0).astype(bf16)

    h_cat = jnp.concatenate([_shift_down(h), h, _shift_up(h)], axis=1)
    g = jnp.maximum(dot(h_cat, wt1b_ref[...]) + bt1r_ref[...], 0.0).astype(bf16)

    ga = g[:, :_D]
    gb = g[:, _D:]
    g_cat = jnp.concatenate([_shift_down(gb), ga, gb, _shift_up(ga)], axis=1)
    y_ref[0] = dot(g_cat, wt2b_ref[...]) + bt2r_ref[...]


def kernel(x, w1, b1, w2, b2, w3, b3, w4, b4, wp, bp, E, wd1, bd1, wt1, bt1,
           wt2, bt2):
    f32 = jnp.float32

    taps1 = w1[:, 0, :].T
    z3_64 = jnp.zeros((3, _D), f32)
    z5_64 = jnp.zeros((5, _D), f32)
    z4_64 = jnp.zeros((4, _D), f32)
    w1b = jnp.concatenate([
        jnp.concatenate([z3_64, taps1, z5_64, z4_64], axis=0),
        jnp.concatenate([z5_64, taps1, z3_64, z4_64], axis=0),
    ], axis=1)
    b1r = jnp.concatenate([b1, b1])[None, :]

    w2b = w2.transpose(2, 1, 0).reshape(256, 128)
    b2r = b2[None, :]

    w3b = w3.transpose(2, 1, 0).reshape(384, 128)
    b3r = b3[None, :]

    w4t = w4[:, :, 0].T
    b4r = b4[None, :]
    wpt = wp[:, :, 0].T
    bpr = bp[None, :]

    et2 = 2.0 * E.T
    e2r = jnp.sum(E * E, axis=1)[None, :]

    wd1b = wd1.transpose(2, 1, 0).reshape(192, 128)
    bd1r = bd1[None, :]

    tt = wt1.transpose(2, 0, 1)[::-1]
    zero = jnp.zeros((128, _D), f32)
    wt1b = jnp.concatenate([
        jnp.concatenate([tt[0], zero], axis=1),
        jnp.concatenate([tt[2], tt[1]], axis=1),
        jnp.concatenate([zero, tt[3]], axis=1),
    ], axis=0)
    bt1r = jnp.concatenate([bt1, bt1])[None, :]

    v = wt2[:, 0, ::-1].T
    z64 = jnp.zeros((_D,), f32)
    z128 = jnp.zeros((2 * _D,), f32)
    wt2b = jnp.stack([
        jnp.concatenate([v[0], v[2], z128]),
        jnp.concatenate([z64, v[1], v[3], z64]),
        jnp.concatenate([z64, v[0], v[2], z64]),
        jnp.concatenate([z128, v[1], v[3]]),
    ], axis=1)
    bt2r = jnp.broadcast_to(bt2[0], (1, 4)).astype(f32)

    bf16 = jnp.bfloat16
    xp = jnp.pad(x.reshape(_B, 16384).astype(bf16), ((0, 0), (4, 12)))
    xr = jnp.concatenate(
        [xp[:, 4 * s:4 * s + 16384].reshape(_B, _P, 4) for s in range(4)],
        axis=2)
    w1b, w2b, w3b, w4t, wpt, eb, wd1b, wt1b, wt2b = (
        t.astype(bf16) for t in (w1b, w2b, w3b, w4t, wpt, E, wd1b, wt1b, wt2b))

    rep2 = lambda shape: pl.BlockSpec(shape, lambda i: (0, 0))
    grid_spec = pl.GridSpec(
        grid=(_B,),
        in_specs=[
            pl.BlockSpec((1, _P, 16), lambda i: (i, 0, 0)),
            rep2((16, 128)), rep2((1, 128)),
            rep2((256, 128)), rep2((1, 128)),
            rep2((384, 128)), rep2((1, 128)),
            rep2((128, 64)), rep2((1, 64)),
            rep2((64, 64)), rep2((1, 64)),
            rep2((64, _K)), rep2((1, _K)), rep2((_K, _D)),
            rep2((192, 128)), rep2((1, 128)),
            rep2((384, 128)), rep2((1, 128)),
            rep2((256, 4)), rep2((1, 4)),
        ],
        out_specs=[
            pl.BlockSpec((1, _P, 4), lambda i: (i, 0, 0)),
            pl.BlockSpec((1, _P, 1), lambda i: (i, 0, 0)),
            pl.BlockSpec((1, 1, _D), lambda i: (i, 0, 0)),
        ],
    )
    y4, idx, lp = pl.pallas_call(
        _vqvae_body,
        grid_spec=grid_spec,
        out_shape=[
            jax.ShapeDtypeStruct((_B, _P, 4), f32),
            jax.ShapeDtypeStruct((_B, _P, 1), jnp.int32),
            jax.ShapeDtypeStruct((_B, 1, _D), f32),
        ],
    )(xr, w1b, b1r, w2b, b2r, w3b, b3r, w4t, b4r, wpt, bpr, et2, e2r, eb,
      wd1b, bd1r, wt1b, bt1r, wt2b, bt2r)

    loss = jnp.sum(lp) * (1.25 / (_B * _P * _D))
    y = y4.reshape(_B, 1, 16384)
    return (loss, y, idx.reshape(_B * _P, 1))


# speedup vs baseline: 1.3548x; 1.3548x over previous
import functools

import jax
import jax.numpy as jnp
from jax.experimental import pallas as pl
from jax.experimental.pallas import tpu as pltpu

_B = 32
_P = 4096
_K = 512
_D = 64


def _shift_down(z):
    c = z.shape[1]
    return jnp.concatenate([jnp.zeros((1, c), z.dtype), z[:-1, :]], axis=0)


def _shift_up(z):
    c = z.shape[1]
    return jnp.concatenate([z[1:, :], jnp.zeros((1, c), z.dtype)], axis=0)


def _vqvae_body(x_ref, w1b_ref, b1r_ref, w2b_ref, b2r_ref, w3b_ref, b3r_ref,
                w4t_ref, b4r_ref, wpt_ref, bpr_ref,
                et2_ref, e2r_ref, e_ref, wd1b_ref, bd1r_ref,
                wt1b_ref, bt1r_ref, wt2b_ref, bt2r_ref,
                y_ref, idx_ref, lp_ref):
    f32 = jnp.float32
    bf16 = jnp.bfloat16
    dot = functools.partial(jnp.dot, preferred_element_type=f32)

    for _bb in range(2):
      xq = x_ref[_bb]
      x_cat = jnp.concatenate([_shift_down(xq), xq, _shift_up(xq)], axis=1)
      z1 = jnp.maximum(dot(x_cat, w1b_ref[...]) + b1r_ref[...], 0.0).astype(bf16)

      a = z1[:, :_D]
      b = z1[:, _D:]
      z_cat = jnp.concatenate([_shift_down(b), a, b, _shift_up(a)], axis=1)
      z2 = jnp.maximum(dot(z_cat, w2b_ref[...]) + b2r_ref[...], 0.0).astype(bf16)

      z_cat = jnp.concatenate([_shift_down(z2), z2, _shift_up(z2)], axis=1)
      z3 = jnp.maximum(dot(z_cat, w3b_ref[...]) + b3r_ref[...], 0.0).astype(bf16)

      z4 = (dot(z3, w4t_ref[...]) + b4r_ref[...]).astype(bf16)
      flat = dot(z4, wpt_ref[...]) + bpr_ref[...]

      flat2 = jnp.sum(flat * flat, axis=1, keepdims=True)
      dist = (flat2 + e2r_ref[...]) - dot(flat, et2_ref[...])
      mval = jnp.min(dist, axis=1, keepdims=True)
      colid = jax.lax.broadcasted_iota(jnp.int32, (_P, _K), 1)
      idx = jnp.min(jnp.where(dist == mval, colid, _K), axis=1, keepdims=True)
      onehot = (colid == idx).astype(bf16)
      q = dot(onehot, e_ref[...])
      idx_ref[_bb] = idx

      diff = q - flat
      lp_ref[_bb] = jnp.sum(diff * diff, axis=0, keepdims=True)

      qb = q.astype(bf16)
      q_cat = jnp.concatenate([_shift_down(qb), qb, _shift_up(qb)], axis=1)
      h = jnp.maximum(dot(q_cat, wd1b_ref[...]) + bd1r_ref[...], 0.0).astype(bf16)

      h_cat = jnp.concatenate([_shift_down(h), h, _shift_up(h)], axis=1)
      g = jnp.maximum(dot(h_cat, wt1b_ref[...]) + bt1r_ref[...], 0.0).astype(bf16)

      ga = g[:, :_D]
      gb = g[:, _D:]
      g_cat = jnp.concatenate([_shift_down(gb), ga, gb, _shift_up(ga)], axis=1)
      y_ref[_bb] = dot(g_cat, wt2b_ref[...]) + bt2r_ref[...]


def kernel(x, w1, b1, w2, b2, w3, b3, w4, b4, wp, bp, E, wd1, bd1, wt1, bt1,
           wt2, bt2):
    f32 = jnp.float32

    taps1 = w1[:, 0, :].T
    z3_64 = jnp.zeros((3, _D), f32)
    z5_64 = jnp.zeros((5, _D), f32)
    w1b = jnp.concatenate([
        jnp.concatenate([z3_64, taps1, z5_64], axis=0),
        jnp.concatenate([z5_64, taps1, z3_64], axis=0),
    ], axis=1)
    b1r = jnp.concatenate([b1, b1])[None, :]

    w2b = w2.transpose(2, 1, 0).reshape(256, 128)
    b2r = b2[None, :]

    w3b = w3.transpose(2, 1, 0).reshape(384, 128)
    b3r = b3[None, :]

    w4t = w4[:, :, 0].T
    b4r = b4[None, :]
    wpt = wp[:, :, 0].T
    bpr = bp[None, :]

    et2 = 2.0 * E.T
    e2r = jnp.sum(E * E, axis=1)[None, :]

    wd1b = wd1.transpose(2, 1, 0).reshape(192, 128)
    bd1r = bd1[None, :]

    tt = wt1.transpose(2, 0, 1)[::-1]
    zero = jnp.zeros((128, _D), f32)
    wt1b = jnp.concatenate([
        jnp.concatenate([tt[0], zero], axis=1),
        jnp.concatenate([tt[2], tt[1]], axis=1),
        jnp.concatenate([zero, tt[3]], axis=1),
    ], axis=0)
    bt1r = jnp.concatenate([bt1, bt1])[None, :]

    v = wt2[:, 0, ::-1].T
    z64 = jnp.zeros((_D,), f32)
    z128 = jnp.zeros((2 * _D,), f32)
    wt2b = jnp.stack([
        jnp.concatenate([v[0], v[2], z128]),
        jnp.concatenate([z64, v[1], v[3], z64]),
        jnp.concatenate([z64, v[0], v[2], z64]),
        jnp.concatenate([z128, v[1], v[3]]),
    ], axis=1)
    bt2r = jnp.broadcast_to(bt2[0], (1, 4)).astype(f32)

    bf16 = jnp.bfloat16
    xr = x.reshape(_B, _P, 4).astype(bf16)
    w1b, w2b, w3b, w4t, wpt, eb, wd1b, wt1b, wt2b = (
        t.astype(bf16) for t in (w1b, w2b, w3b, w4t, wpt, E, wd1b, wt1b, wt2b))

    rep2 = lambda shape: pl.BlockSpec(shape, lambda i: (0, 0))
    grid_spec = pl.GridSpec(
        grid=(_B // 2,),
        in_specs=[
            pl.BlockSpec((2, _P, 4), lambda i: (i, 0, 0)),
            rep2((12, 128)), rep2((1, 128)),
            rep2((256, 128)), rep2((1, 128)),
            rep2((384, 128)), rep2((1, 128)),
            rep2((128, 64)), rep2((1, 64)),
            rep2((64, 64)), rep2((1, 64)),
            rep2((64, _K)), rep2((1, _K)), rep2((_K, _D)),
            rep2((192, 128)), rep2((1, 128)),
            rep2((384, 128)), rep2((1, 128)),
            rep2((256, 4)), rep2((1, 4)),
        ],
        out_specs=[
            pl.BlockSpec((2, _P, 4), lambda i: (i, 0, 0)),
            pl.BlockSpec((2, _P, 1), lambda i: (i, 0, 0)),
            pl.BlockSpec((2, 1, _D), lambda i: (i, 0, 0)),
        ],
    )
    y4, idx, lp = pl.pallas_call(
        _vqvae_body,
        grid_spec=grid_spec,
        out_shape=[
            jax.ShapeDtypeStruct((_B, _P, 4), f32),
            jax.ShapeDtypeStruct((_B, _P, 1), jnp.int32),
            jax.ShapeDtypeStruct((_B, 1, _D), f32),
        ],
    )(xr, w1b, b1r, w2b, b2r, w3b, b3r, w4t, b4r, wpt, bpr, et2, e2r, eb,
      wd1b, bd1r, wt1b, bt1r, wt2b, bt2r)

    loss = jnp.sum(lp) * (1.25 / (_B * _P * _D))
    y = y4.reshape(_B, 1, 16384)
    return (loss, y, idx.reshape(_B * _P, 1))


# speedup vs baseline: 1.5906x; 1.1740x over previous
import functools

import jax
import jax.numpy as jnp
from jax.experimental import pallas as pl
from jax.experimental.pallas import tpu as pltpu

_B = 32
_P = 4096
_K = 512
_D = 64


def _shift_down(z):
    c = z.shape[1]
    return jnp.concatenate([jnp.zeros((1, c), z.dtype), z[:-1, :]], axis=0)


def _shift_up(z):
    c = z.shape[1]
    return jnp.concatenate([z[1:, :], jnp.zeros((1, c), z.dtype)], axis=0)


def _vqvae_body(x_ref, w1b_ref, b1r_ref, w2b_ref, b2r_ref, w3b_ref, b3r_ref,
                w4t_ref, b4r_ref, wpt_ref, bpr_ref,
                et2_ref, e2r_ref, e_ref, wd1b_ref, bd1r_ref,
                wt1b_ref, bt1r_ref, wt2b_ref, bt2r_ref,
                y_ref, idx_ref, lp_ref):
    f32 = jnp.float32
    bf16 = jnp.bfloat16
    dot = functools.partial(jnp.dot, preferred_element_type=f32)

    for _bb in range(2):
      xq = x_ref[_bb]
      x_cat = jnp.concatenate([_shift_down(xq), xq, _shift_up(xq)], axis=1)
      z1 = jnp.maximum(dot(x_cat, w1b_ref[...]) + b1r_ref[...], 0.0).astype(bf16)

      a = z1[:, :_D]
      b = z1[:, _D:]
      z_cat = jnp.concatenate([_shift_down(b), a, b, _shift_up(a)], axis=1)
      z2 = jnp.maximum(dot(z_cat, w2b_ref[...]) + b2r_ref[...], 0.0).astype(bf16)

      z_cat = jnp.concatenate([_shift_down(z2), z2, _shift_up(z2)], axis=1)
      z3 = jnp.maximum(dot(z_cat, w3b_ref[...]) + b3r_ref[...], 0.0).astype(bf16)

      z4 = (dot(z3, w4t_ref[...]) + b4r_ref[...]).astype(bf16)
      flat = dot(z4, wpt_ref[...]) + bpr_ref[...]

      flat2 = jnp.sum(flat * flat, axis=1, keepdims=True)
      dist = (flat2 + e2r_ref[...]) - dot(flat, et2_ref[...])
      mval = jnp.min(dist, axis=1, keepdims=True)
      colid = jax.lax.broadcasted_iota(jnp.int32, (_P, _K), 1)
      idx = jnp.min(jnp.where(dist == mval, colid, _K), axis=1, keepdims=True)
      onehot = (colid == idx).astype(bf16)
      q = dot(onehot, e_ref[...])
      idx_ref[_bb] = idx.T

      diff = q - flat
      lp_ref[_bb] = jnp.sum(diff * diff, axis=0, keepdims=True)

      qb = q.astype(bf16)
      q_cat = jnp.concatenate([_shift_down(qb), qb, _shift_up(qb)], axis=1)
      h = jnp.maximum(dot(q_cat, wd1b_ref[...]) + bd1r_ref[...], 0.0).astype(bf16)

      h_cat = jnp.concatenate([_shift_down(h), h, _shift_up(h)], axis=1)
      g = jnp.maximum(dot(h_cat, wt1b_ref[...]) + bt1r_ref[...], 0.0).astype(bf16)

      ga = g[:, :_D]
      gb = g[:, _D:]
      g_cat = jnp.concatenate([_shift_down(gb), ga, gb, _shift_up(ga)], axis=1)
      y = dot(g_cat, wt2b_ref[...]) + bt2r_ref[...]
      y_ref[_bb] = y.T


def kernel(x, w1, b1, w2, b2, w3, b3, w4, b4, wp, bp, E, wd1, bd1, wt1, bt1,
           wt2, bt2):
    f32 = jnp.float32

    taps1 = w1[:, 0, :].T
    z3_64 = jnp.zeros((3, _D), f32)
    z5_64 = jnp.zeros((5, _D), f32)
    w1b = jnp.concatenate([
        jnp.concatenate([z3_64, taps1, z5_64], axis=0),
        jnp.concatenate([z5_64, taps1, z3_64], axis=0),
    ], axis=1)
    b1r = jnp.concatenate([b1, b1])[None, :]

    w2b = w2.transpose(2, 1, 0).reshape(256, 128)
    b2r = b2[None, :]

    w3b = w3.transpose(2, 1, 0).reshape(384, 128)
    b3r = b3[None, :]

    w4t = w4[:, :, 0].T
    b4r = b4[None, :]
    wpt = wp[:, :, 0].T
    bpr = bp[None, :]

    et2 = 2.0 * E.T
    e2r = jnp.sum(E * E, axis=1)[None, :]

    wd1b = wd1.transpose(2, 1, 0).reshape(192, 128)
    bd1r = bd1[None, :]

    tt = wt1.transpose(2, 0, 1)[::-1]
    zero = jnp.zeros((128, _D), f32)
    wt1b = jnp.concatenate([
        jnp.concatenate([tt[0], zero], axis=1),
        jnp.concatenate([tt[2], tt[1]], axis=1),
        jnp.concatenate([zero, tt[3]], axis=1),
    ], axis=0)
    bt1r = jnp.concatenate([bt1, bt1])[None, :]

    v = wt2[:, 0, ::-1].T
    z64 = jnp.zeros((_D,), f32)
    z128 = jnp.zeros((2 * _D,), f32)
    wt2b = jnp.stack([
        jnp.concatenate([v[0], v[2], z128]),
        jnp.concatenate([z64, v[1], v[3], z64]),
        jnp.concatenate([z64, v[0], v[2], z64]),
        jnp.concatenate([z128, v[1], v[3]]),
    ], axis=1)
    bt2r = jnp.broadcast_to(bt2[0], (1, 4)).astype(f32)

    bf16 = jnp.bfloat16
    xr = x.reshape(_B, _P, 4).astype(bf16)
    w1b, w2b, w3b, w4t, wpt, eb, wd1b, wt1b, wt2b = (
        t.astype(bf16) for t in (w1b, w2b, w3b, w4t, wpt, E, wd1b, wt1b, wt2b))

    rep2 = lambda shape: pl.BlockSpec(shape, lambda i: (0, 0))
    grid_spec = pl.GridSpec(
        grid=(_B // 2,),
        in_specs=[
            pl.BlockSpec((2, _P, 4), lambda i: (i, 0, 0)),
            rep2((12, 128)), rep2((1, 128)),
            rep2((256, 128)), rep2((1, 128)),
            rep2((384, 128)), rep2((1, 128)),
            rep2((128, 64)), rep2((1, 64)),
            rep2((64, 64)), rep2((1, 64)),
            rep2((64, _K)), rep2((1, _K)), rep2((_K, _D)),
            rep2((192, 128)), rep2((1, 128)),
            rep2((384, 128)), rep2((1, 128)),
            rep2((256, 4)), rep2((1, 4)),
        ],
        out_specs=[
            pl.BlockSpec((2, 4, _P), lambda i: (i, 0, 0)),
            pl.BlockSpec((2, 1, _P), lambda i: (i, 0, 0)),
            pl.BlockSpec((2, 1, _D), lambda i: (i, 0, 0)),
        ],
    )
    y4, idx, lp = pl.pallas_call(
        _vqvae_body,
        grid_spec=grid_spec,
        out_shape=[
            jax.ShapeDtypeStruct((_B, 4, _P), f32),
            jax.ShapeDtypeStruct((_B, 1, _P), jnp.int32),
            jax.ShapeDtypeStruct((_B, 1, _D), f32),
        ],
    )(xr, w1b, b1r, w2b, b2r, w3b, b3r, w4t, b4r, wpt, bpr, et2, e2r, eb,
      wd1b, bd1r, wt1b, bt1r, wt2b, bt2r)

    loss = jnp.sum(lp) * (1.25 / (_B * _P * _D))
    y = y4.transpose(0, 2, 1).reshape(_B, 1, 16384)
    return (loss, y, idx.reshape(_B * _P, 1))


# speedup vs baseline: 1.6424x; 1.0326x over previous
import functools

import jax
import jax.numpy as jnp
from jax.experimental import pallas as pl
from jax.experimental.pallas import tpu as pltpu

_B = 32
_P = 4096
_K = 512
_D = 64


def _shift_down(z):
    c = z.shape[1]
    return jnp.concatenate([jnp.zeros((1, c), z.dtype), z[:-1, :]], axis=0)


def _shift_up(z):
    c = z.shape[1]
    return jnp.concatenate([z[1:, :], jnp.zeros((1, c), z.dtype)], axis=0)


def _vqvae_body(x_ref, w1b_ref, b1r_ref, w2b_ref, b2r_ref, w3b_ref, b3r_ref,
                w4t_ref, b4r_ref, wpt_ref, bpr_ref,
                et2_ref, e2r_ref, e_ref, wd1b_ref, bd1r_ref,
                wt1b_ref, bt1r_ref, wt2b_ref, bt2r_ref,
                y_ref, idx_ref, lp_ref):
    f32 = jnp.float32
    bf16 = jnp.bfloat16
    dot = functools.partial(jnp.dot, preferred_element_type=f32)

    for _bb in range(2):
      xq = x_ref[_bb].T
      x_cat = jnp.concatenate([_shift_down(xq), xq, _shift_up(xq)], axis=1)
      z1 = jnp.maximum(dot(x_cat, w1b_ref[...]) + b1r_ref[...], 0.0).astype(bf16)

      a = z1[:, :_D]
      b = z1[:, _D:]
      z_cat = jnp.concatenate([_shift_down(b), a, b, _shift_up(a)], axis=1)
      z2 = jnp.maximum(dot(z_cat, w2b_ref[...]) + b2r_ref[...], 0.0).astype(bf16)

      z_cat = jnp.concatenate([_shift_down(z2), z2, _shift_up(z2)], axis=1)
      z3 = jnp.maximum(dot(z_cat, w3b_ref[...]) + b3r_ref[...], 0.0).astype(bf16)

      z4 = (dot(z3, w4t_ref[...]) + b4r_ref[...]).astype(bf16)
      flat = dot(z4, wpt_ref[...]) + bpr_ref[...]

      flat2 = jnp.sum(flat * flat, axis=1, keepdims=True)
      dist = (flat2 + e2r_ref[...]) - dot(flat, et2_ref[...])
      mval = jnp.min(dist, axis=1, keepdims=True)
      colid = jax.lax.broadcasted_iota(jnp.int32, (_P, _K), 1)
      idx = jnp.min(jnp.where(dist == mval, colid, _K), axis=1, keepdims=True)
      onehot = (colid == idx).astype(bf16)
      q = dot(onehot, e_ref[...])
      idx_ref[_bb] = idx.T

      diff = q - flat
      lp_ref[_bb] = jnp.sum(diff * diff, axis=0, keepdims=True)

      qb = q.astype(bf16)
      q_cat = jnp.concatenate([_shift_down(qb), qb, _shift_up(qb)], axis=1)
      h = jnp.maximum(dot(q_cat, wd1b_ref[...]) + bd1r_ref[...], 0.0).astype(bf16)

      h_cat = jnp.concatenate([_shift_down(h), h, _shift_up(h)], axis=1)
      g = jnp.maximum(dot(h_cat, wt1b_ref[...]) + bt1r_ref[...], 0.0).astype(bf16)

      ga = g[:, :_D]
      gb = g[:, _D:]
      g_cat = jnp.concatenate([_shift_down(gb), ga, gb, _shift_up(ga)], axis=1)
      y = dot(g_cat, wt2b_ref[...]) + bt2r_ref[...]
      y_ref[_bb] = y.T


def kernel(x, w1, b1, w2, b2, w3, b3, w4, b4, wp, bp, E, wd1, bd1, wt1, bt1,
           wt2, bt2):
    f32 = jnp.float32

    taps1 = w1[:, 0, :].T
    z3_64 = jnp.zeros((3, _D), f32)
    z5_64 = jnp.zeros((5, _D), f32)
    w1b = jnp.concatenate([
        jnp.concatenate([z3_64, taps1, z5_64], axis=0),
        jnp.concatenate([z5_64, taps1, z3_64], axis=0),
    ], axis=1)
    b1r = jnp.concatenate([b1, b1])[None, :]

    w2b = w2.transpose(2, 1, 0).reshape(256, 128)
    b2r = b2[None, :]

    w3b = w3.transpose(2, 1, 0).reshape(384, 128)
    b3r = b3[None, :]

    w4t = w4[:, :, 0].T
    b4r = b4[None, :]
    wpt = wp[:, :, 0].T
    bpr = bp[None, :]

    et2 = 2.0 * E.T
    e2r = jnp.sum(E * E, axis=1)[None, :]

    wd1b = wd1.transpose(2, 1, 0).reshape(192, 128)
    bd1r = bd1[None, :]

    tt = wt1.transpose(2, 0, 1)[::-1]
    zero = jnp.zeros((128, _D), f32)
    wt1b = jnp.concatenate([
        jnp.concatenate([tt[0], zero], axis=1),
        jnp.concatenate([tt[2], tt[1]], axis=1),
        jnp.concatenate([zero, tt[3]], axis=1),
    ], axis=0)
    bt1r = jnp.concatenate([bt1, bt1])[None, :]

    v = wt2[:, 0, ::-1].T
    z64 = jnp.zeros((_D,), f32)
    z128 = jnp.zeros((2 * _D,), f32)
    wt2b = jnp.stack([
        jnp.concatenate([v[0], v[2], z128]),
        jnp.concatenate([z64, v[1], v[3], z64]),
        jnp.concatenate([z64, v[0], v[2], z64]),
        jnp.concatenate([z128, v[1], v[3]]),
    ], axis=1)
    bt2r = jnp.broadcast_to(bt2[0], (1, 4)).astype(f32)

    bf16 = jnp.bfloat16
    xr = x.reshape(_B, _P, 4).transpose(0, 2, 1).astype(bf16)
    w1b, w2b, w3b, w4t, wpt, eb, wd1b, wt1b, wt2b = (
        t.astype(bf16) for t in (w1b, w2b, w3b, w4t, wpt, E, wd1b, wt1b, wt2b))

    rep2 = lambda shape: pl.BlockSpec(shape, lambda i: (0, 0))
    grid_spec = pl.GridSpec(
        grid=(_B // 2,),
        in_specs=[
            pl.BlockSpec((2, 4, _P), lambda i: (i, 0, 0)),
            rep2((12, 128)), rep2((1, 128)),
            rep2((256, 128)), rep2((1, 128)),
            rep2((384, 128)), rep2((1, 128)),
            rep2((128, 64)), rep2((1, 64)),
            rep2((64, 64)), rep2((1, 64)),
            rep2((64, _K)), rep2((1, _K)), rep2((_K, _D)),
            rep2((192, 128)), rep2((1, 128)),
            rep2((384, 128)), rep2((1, 128)),
            rep2((256, 4)), rep2((1, 4)),
        ],
        out_specs=[
            pl.BlockSpec((2, 4, _P), lambda i: (i, 0, 0)),
            pl.BlockSpec((2, 1, _P), lambda i: (i, 0, 0)),
            pl.BlockSpec((2, 1, _D), lambda i: (i, 0, 0)),
        ],
    )
    y4, idx, lp = pl.pallas_call(
        _vqvae_body,
        grid_spec=grid_spec,
        out_shape=[
            jax.ShapeDtypeStruct((_B, 4, _P), f32),
            jax.ShapeDtypeStruct((_B, 1, _P), jnp.int32),
            jax.ShapeDtypeStruct((_B, 1, _D), f32),
        ],
    )(xr, w1b, b1r, w2b, b2r, w3b, b3r, w4t, b4r, wpt, bpr, et2, e2r, eb,
      wd1b, bd1r, wt1b, bt1r, wt2b, bt2r)

    loss = jnp.sum(lp) * (1.25 / (_B * _P * _D))
    y = y4.transpose(0, 2, 1).reshape(_B, 1, 16384)
    return (loss, y, idx.reshape(_B * _P, 1))
